# baseline (speedup 1.0000x reference)
import functools

import jax
import jax.numpy as jnp
from jax import lax
from jax.experimental import pallas as pl
from jax.experimental.pallas import tpu as pltpu
from jax.experimental.pallas import tpu_sc as plsc

_NUM_CLASSES = 3
_NUM_ANCHORS = 3
_EXCEPT_THRESH = 0.05
_NMS_THRESH = 0.5

_B = 2
_N = 48384
_TOT = _B * _N

_IBLK = 256
_NIBLK = _N // _IBLK
_JCHUNK = 2688
_NJCHUNK = _N // _JCHUNK

_NW = 32
_PER_TILE = _TOT // _NW
_IDXC = 112
_NCHUNK = _PER_TILE // _IDXC


def _decode(output, anchor, offset, stride):
    B, H, W, _ = output.shape
    out = output.reshape(B, H, W, _NUM_ANCHORS, 5 + _NUM_CLASSES)
    txty = jax.nn.sigmoid(out[..., 0:2])
    twth = out[..., 2:4]
    obj = jax.nn.sigmoid(out[..., 4:5])
    cls = jax.nn.sigmoid(out[..., 5:])
    cxcy = (txty + offset) * stride
    wh = jnp.exp(twth) * anchor
    half = wh * 0.5
    xmin = cxcy[..., 0:1] - half[..., 0:1]
    ymin = cxcy[..., 1:2] - half[..., 1:2]
    xmax = cxcy[..., 0:1] + half[..., 0:1]
    ymax = cxcy[..., 1:2] + half[..., 1:2]
    score = obj * cls
    cid = jnp.broadcast_to(jnp.arange(_NUM_CLASSES, dtype=jnp.float32), score.shape)
    cid = jnp.where(score > _EXCEPT_THRESH, cid, -jnp.ones_like(cid))
    box = jnp.concatenate([xmin, ymin, xmax, ymax], axis=-1)
    box = jnp.broadcast_to(box[..., None, :], score.shape + (4,))
    res = jnp.concatenate([cid[..., None], score[..., None], box], axis=-1)
    return res.reshape(B, -1, 6)


def _rank_body(col_ref, row_ref, out_ref):
    b = pl.program_id(0)
    blk = pl.program_id(1)
    key_col = col_ref[0, 0]
    i_col = blk * _IBLK + lax.broadcasted_iota(jnp.int32, (_IBLK, 1), 0)

    def body(c, acc):
        keys_j = row_ref[0, :, pl.ds(c * _JCHUNK, _JCHUNK)]
        j_row = c * _JCHUNK + lax.broadcasted_iota(jnp.int32, (1, _JCHUNK), 1)
        lt = (keys_j < key_col) | ((keys_j == key_col) & (j_row < i_col))
        return acc + jnp.sum(lt.astype(jnp.int32), axis=1, keepdims=True)

    rank = lax.fori_loop(0, _NJCHUNK, body, jnp.zeros((_IBLK, 1), jnp.int32))
    out_ref[0, 0] = rank + b * _N


def _compute_dest(ikey):
    col = ikey.reshape(_B, _NIBLK, _IBLK, 1)
    row = ikey.reshape(_B, 1, _N)
    dest = pl.pallas_call(
        _rank_body,
        grid=(_B, _NIBLK),
        in_specs=[
            pl.BlockSpec((1, 1, _IBLK, 1), lambda b, i: (b, i, 0, 0)),
            pl.BlockSpec((1, 1, _N), lambda b, i: (b, 0, 0)),
        ],
        out_specs=pl.BlockSpec((1, 1, _IBLK, 1), lambda b, i: (b, i, 0, 0)),
        out_shape=jax.ShapeDtypeStruct((_B, _NIBLK, _IBLK, 1), jnp.int32),
    )(col, row)
    return dest.reshape(_TOT)


@functools.cache
def _build_sc_scatter():
    @functools.partial(
        pl.kernel,
        mesh=plsc.VectorSubcoreMesh(
            core_axis_name="c", subcore_axis_name="s",
            num_cores=2, num_subcores=16),
        out_type=jax.ShapeDtypeStruct((_TOT, 8), jnp.float32),
        compiler_params=pltpu.CompilerParams(use_tc_tiling_on_sc=False),
        scratch_types=[
            pltpu.VMEM((_NCHUNK, _IDXC), jnp.int32),
            pltpu.VMEM((_PER_TILE, 8), jnp.float32),
            pltpu.SemaphoreType.DMA,
        ],
    )
    def _sc_scatter(dest_hbm, x_hbm, out_hbm, idx_v, rows_v, sem):
        cid = lax.axis_index("c")
        sid = lax.axis_index("s")
        wid = sid * 2 + cid
        pltpu.sync_copy(dest_hbm.at[pl.ds(wid * _NCHUNK, _NCHUNK)], idx_v)
        pltpu.sync_copy(x_hbm.at[pl.ds(wid * _PER_TILE, _PER_TILE)], rows_v)
        for j in range(_NCHUNK):
            pltpu.async_copy(
                rows_v.at[pl.ds(j * _IDXC, _IDXC)],
                out_hbm.at[idx_v.at[j]],
                sem,
            ).wait()

    return _sc_scatter


def _nms_body(t_ref, out_ref):
    t = t_ref[...]
    g = t[0:1]
    s = t[1:2]
    bx = t[2:6]
    prev = jnp.concatenate([t[:, 0:1], t[:, :-1]], axis=1)
    gp = prev[0:1]
    x1p, y1p, x2p, y2p = prev[2:3], prev[3:4], prev[4:5], prev[5:6]
    x1, y1, x2, y2 = t[2:3], t[3:4], t[4:5], t[5:6]
    xx1 = jnp.maximum(x1p, x1)
    yy1 = jnp.maximum(y1p, y1)
    xx2 = jnp.minimum(x2p, x2)
    yy2 = jnp.minimum(y2p, y2)
    w = xx2 - xx1 + 1.0
    h = yy2 - yy1 + 1.0
    area_p = (x2p - x1p + 1.0) * (y2p - y1p + 1.0)
    area_n = (x2 - x1 + 1.0) * (y2 - y1 + 1.0)
    inter = w * h
    overlap = inter / (area_p + area_n - inter)
    k = lax.broadcasted_iota(jnp.int32, (1, _TOT), 1)
    first = (k == 0) | (k == _N)
    same = (g == gp) & (g >= 0.0)
    supp = same & (overlap > _NMS_THRESH) & jnp.logical_not(first)
    mask = jnp.where(supp, -1.0, 1.0).astype(jnp.float32)
    in_nms = g >= 0.0
    gm = jnp.where(in_nms, g * mask, g)
    sm = jnp.where(in_nms, s * mask, s)
    bm = jnp.where(in_nms, bx * mask, bx)
    gm = jnp.where(in_nms & (gm < 0.0), -1.0, gm)
    sm = jnp.where(in_nms & (sm < 0.0), -1.0, sm)
    bm = jnp.where(in_nms & (bm < 0.0), -1.0, bm)
    zero = jnp.zeros((2, _TOT), jnp.float32)
    out_ref[...] = jnp.concatenate([gm, sm, bm, zero], axis=0)


def _nms(t):
    return pl.pallas_call(
        _nms_body,
        out_shape=jax.ShapeDtypeStruct((8, _TOT), jnp.float32),
    )(t)


def kernel(output1, output2, output3, anchor1, anchor2, anchor3,
           offset1, offset2, offset3, stride1, stride2, stride3):
    results = jnp.concatenate([
        _decode(output1, anchor1, offset1, stride1),
        _decode(output2, anchor2, offset2, stride2),
        _decode(output3, anchor3, offset3, stride3)], axis=1)
    ids = results[:, :, 0]
    scores = results[:, :, 1]
    boxes = results[:, :, 2:]

    iv = ids.astype(jnp.int32)
    sbits = lax.bitcast_convert_type(scores, jnp.int32)
    idx = jnp.broadcast_to(jnp.arange(_N, dtype=jnp.int32), (_B, _N))
    secint = jnp.where(iv < 0, idx, (1 << 30) - sbits)
    ukey = ((iv + 1).astype(jnp.uint32) << 30) + secint.astype(jnp.uint32)
    ikey = lax.bitcast_convert_type(ukey ^ jnp.uint32(0x80000000), jnp.int32)

    dest = _compute_dest(ikey)
    dest2d = dest.reshape(_NW * _NCHUNK, _IDXC)

    pay = jnp.concatenate(
        [ids[..., None], scores[..., None], boxes,
         jnp.zeros((_B, _N, 2), jnp.float32)], axis=-1).reshape(_TOT, 8)

    sorted8 = _build_sc_scatter()(dest2d, pay)

    o = _nms(sorted8.T)

    gm = o[0].reshape(_B, _N)[:, :, None]
    sm = o[1].reshape(_B, _N)[:, :, None]
    bm = o[2:6].reshape(4, _B, _N).transpose(1, 2, 0)
    return gm, sm, bm


# speedup vs baseline: 7.1695x; 7.1695x over previous
import functools

import jax
import jax.numpy as jnp
from jax import lax
from jax.experimental import pallas as pl
from jax.experimental.pallas import tpu as pltpu
from jax.experimental.pallas import tpu_sc as plsc

_NUM_CLASSES = 3
_NUM_ANCHORS = 3
_EXCEPT_THRESH = 0.05
_NMS_THRESH = 0.5

_B = 2
_N = 48384
_TOT = _B * _N
_NP = 49152
_TOTP = _B * _NP

_L = 256
_NCH = _NP // _L
_NSLAB = _NCH // 8
_BINS = 256

_NW = 32
_PER_TILE = _TOTP // _NW
_IDXC = 128
_NCHUNK = _PER_TILE // _IDXC


def _decode(output, anchor, offset, stride):
    B, H, W, _ = output.shape
    out = output.reshape(B, H, W, _NUM_ANCHORS, 5 + _NUM_CLASSES)
    txty = jax.nn.sigmoid(out[..., 0:2])
    twth = out[..., 2:4]
    obj = jax.nn.sigmoid(out[..., 4:5])
    cls = jax.nn.sigmoid(out[..., 5:])
    cxcy = (txty + offset) * stride
    wh = jnp.exp(twth) * anchor
    half = wh * 0.5
    xmin = cxcy[..., 0:1] - half[..., 0:1]
    ymin = cxcy[..., 1:2] - half[..., 1:2]
    xmax = cxcy[..., 0:1] + half[..., 0:1]
    ymax = cxcy[..., 1:2] + half[..., 1:2]
    score = obj * cls
    cid = jnp.broadcast_to(jnp.arange(_NUM_CLASSES, dtype=jnp.float32), score.shape)
    cid = jnp.where(score > _EXCEPT_THRESH, cid, -jnp.ones_like(cid))
    box = jnp.concatenate([xmin, ymin, xmax, ymax], axis=-1)
    box = jnp.broadcast_to(box[..., None, :], score.shape + (4,))
    res = jnp.concatenate([cid[..., None], score[..., None], box], axis=-1)
    return res.reshape(B, -1, 6)


def _digit_pass_body(shift, k_ref, out_ref):
    b = pl.program_id(0)
    bins_col = lax.broadcasted_iota(jnp.int32, (_BINS, 1), 0)
    e_r = lax.broadcasted_iota(jnp.int32, (_L, 1), 0)
    e_c = lax.broadcasted_iota(jnp.int32, (1, _L), 1)
    triu_e = (e_r < e_c).astype(jnp.float32)

    def onehot(slab, k):
        keyf = slab[k:k + 1, :]
        ki = lax.bitcast_convert_type(keyf, jnp.int32)
        d = (ki >> shift) & 255
        return (d == bins_col).astype(jnp.float32)

    def hist_step(j, hist):
        slab = k_ref[0, pl.ds(j * 8, 8), :]
        for k in range(8):
            hist = hist + jnp.sum(onehot(slab, k), axis=1, keepdims=True)
        return hist

    hist = lax.fori_loop(0, _NSLAB, hist_step,
                         jnp.zeros((_BINS, 1), jnp.float32))
    inc = hist
    sdist = 1
    while sdist < _BINS:
        inc = inc + jnp.concatenate(
            [jnp.zeros((sdist, 1), jnp.float32), inc[:-sdist]], axis=0)
        sdist *= 2
    base = inc - hist

    def dest_step(j, carry):
        slab = k_ref[0, pl.ds(j * 8, 8), :]
        rows = []
        for k in range(8):
            x = onehot(slab, k)
            p = lax.dot_general(x, triu_e, (((1,), (0,)), ((), ())),
                                preferred_element_type=jnp.float32)
            tot = p + (carry + base)
            rows.append(jnp.sum(tot * x, axis=0, keepdims=True))
            carry = carry + jnp.sum(x, axis=1, keepdims=True)
        dest = jnp.concatenate(rows, axis=0).astype(jnp.int32) + b * _NP
        out_ref[0, pl.ds(j * 8, 8), :] = dest
        return carry

    lax.fori_loop(0, _NSLAB, dest_step, jnp.zeros((_BINS, 1), jnp.float32))


def _digit_pass(keys, shift):
    return pl.pallas_call(
        functools.partial(_digit_pass_body, shift),
        grid=(_B,),
        in_specs=[pl.BlockSpec((1, _NCH, _L), lambda b: (b, 0, 0))],
        out_specs=pl.BlockSpec((1, _NCH, _L), lambda b: (b, 0, 0)),
        out_shape=jax.ShapeDtypeStruct((_B, _NCH, _L), jnp.int32),
    )(keys)


@functools.cache
def _build_sc_scatter():
    @functools.partial(
        pl.kernel,
        mesh=plsc.VectorSubcoreMesh(
            core_axis_name="c", subcore_axis_name="s",
            num_cores=2, num_subcores=16),
        out_type=jax.ShapeDtypeStruct((_TOTP, 8), jnp.float32),
        compiler_params=pltpu.CompilerParams(use_tc_tiling_on_sc=False),
        scratch_types=[
            pltpu.VMEM((_NCHUNK, _IDXC), jnp.int32),
            pltpu.VMEM((_PER_TILE, 8), jnp.float32),
            pltpu.SemaphoreType.DMA,
        ],
    )
    def _sc_scatter(dest_hbm, x_hbm, out_hbm, idx_v, rows_v, sem):
        cid = lax.axis_index("c")
        sid = lax.axis_index("s")
        wid = sid * 2 + cid
        pltpu.sync_copy(dest_hbm.at[pl.ds(wid * _NCHUNK, _NCHUNK)], idx_v)
        pltpu.sync_copy(x_hbm.at[pl.ds(wid * _PER_TILE, _PER_TILE)], rows_v)
        for j in range(_NCHUNK):
            pltpu.async_copy(
                rows_v.at[pl.ds(j * _IDXC, _IDXC)],
                out_hbm.at[idx_v.at[j]],
                sem,
            ).wait()

    return _sc_scatter


def _nms_body(t_ref, out_ref):
    t = t_ref[...]
    g = t[0:1]
    s = t[1:2]
    bx = t[2:6]
    prev = jnp.concatenate([t[:, 0:1], t[:, :-1]], axis=1)
    gp = prev[0:1]
    x1p, y1p, x2p, y2p = prev[2:3], prev[3:4], prev[4:5], prev[5:6]
    x1, y1, x2, y2 = t[2:3], t[3:4], t[4:5], t[5:6]
    xx1 = jnp.maximum(x1p, x1)
    yy1 = jnp.maximum(y1p, y1)
    xx2 = jnp.minimum(x2p, x2)
    yy2 = jnp.minimum(y2p, y2)
    w = xx2 - xx1 + 1.0
    h = yy2 - yy1 + 1.0
    area_p = (x2p - x1p + 1.0) * (y2p - y1p + 1.0)
    area_n = (x2 - x1 + 1.0) * (y2 - y1 + 1.0)
    inter = w * h
    overlap = inter / (area_p + area_n - inter)
    k = lax.broadcasted_iota(jnp.int32, (1, _TOT), 1)
    first = (k == 0) | (k == _N)
    same = (g == gp) & (g >= 0.0)
    supp = same & (overlap > _NMS_THRESH) & jnp.logical_not(first)
    mask = jnp.where(supp, -1.0, 1.0).astype(jnp.float32)
    in_nms = g >= 0.0
    gm = jnp.where(in_nms, g * mask, g)
    sm = jnp.where(in_nms, s * mask, s)
    bm = jnp.where(in_nms, bx * mask, bx)
    gm = jnp.where(in_nms & (gm < 0.0), -1.0, gm)
    sm = jnp.where(in_nms & (sm < 0.0), -1.0, sm)
    bm = jnp.where(in_nms & (bm < 0.0), -1.0, bm)
    zero = jnp.zeros((2, _TOT), jnp.float32)
    out_ref[...] = jnp.concatenate([gm, sm, bm, zero], axis=0)


def _nms(t):
    return pl.pallas_call(
        _nms_body,
        out_shape=jax.ShapeDtypeStruct((8, _TOT), jnp.float32),
    )(t)


def kernel(output1, output2, output3, anchor1, anchor2, anchor3,
           offset1, offset2, offset3, stride1, stride2, stride3):
    results = jnp.concatenate([
        _decode(output1, anchor1, offset1, stride1),
        _decode(output2, anchor2, offset2, stride2),
        _decode(output3, anchor3, offset3, stride3)], axis=1)
    ids = results[:, :, 0]
    scores = results[:, :, 1]
    boxes = results[:, :, 2:]

    iv = ids.astype(jnp.int32)
    sbits = lax.bitcast_convert_type(scores, jnp.int32)
    idx = jnp.broadcast_to(jnp.arange(_N, dtype=jnp.int32), (_B, _N))
    secint = jnp.where(iv < 0, idx, (1 << 30) - sbits)
    ukey = ((iv + 1).astype(jnp.uint32) << 30) + secint.astype(jnp.uint32)
    keyf = lax.bitcast_convert_type(ukey, jnp.float32)

    state = jnp.concatenate(
        [ids[..., None], scores[..., None], boxes, keyf[..., None],
         jnp.zeros((_B, _N, 1), jnp.float32)], axis=-1)
    padkey = lax.bitcast_convert_type(jnp.uint32(0xFFFFFFFF), jnp.float32)
    pad = jnp.zeros((_B, _NP - _N, 8), jnp.float32).at[:, :, 6].set(padkey)
    state = jnp.concatenate([state, pad], axis=1)

    scat = _build_sc_scatter()
    for p in range(4):
        keys = state[:, :, 6].reshape(_B, _NCH, _L)
        dest = _digit_pass(keys, 8 * p)
        dest2d = dest.reshape(_NW * _NCHUNK, _IDXC)
        state = scat(dest2d, state.reshape(_TOTP, 8)).reshape(_B, _NP, 8)

    o = _nms(state[:, :_N, :].reshape(_TOT, 8).T)

    gm = o[0].reshape(_B, _N)[:, :, None]
    sm = o[1].reshape(_B, _N)[:, :, None]
    bm = o[2:6].reshape(4, _B, _N).transpose(1, 2, 0)
    return gm, sm, bm
